# Initial kernel scaffold; baseline (speedup 1.0000x reference)
import jax
import jax.numpy as jnp
from jax.experimental import pallas as pl


def kernel(x, edge_index):
    raise NotImplementedError("write your pallas kernel here")


# speedup vs baseline: 9.3668x; 9.3668x over previous
import functools

import jax
import jax.numpy as jnp
from jax import lax
from jax.experimental import pallas as pl
from jax.experimental.pallas import tpu as pltpu
from jax.experimental.pallas import tpu_sc as plsc

N = 10000
NP = 10240
E = 320000
D = 128
DH = 64
NSUB = 16
NT = NP // NSUB
ET = E // NSUB
CH = 80
NCHUNK = ET // CH
BR = 80
NBLK = NT // BR
ZR = 40
NLAYERS = 3
ALPHA = 1.0 / (NLAYERS + 1)


def _rsqrt16(d):
    y = jnp.full((16,), 1.0, jnp.float32)
    for k in range(1, 20):
        y = jnp.where(d >= float(2 ** k), float(2.0 ** (-k / 2.0)), y)
    for _ in range(4):
        y = y * (1.5 - (0.5 * d) * y * y)
    return y


def _sc_body(x_ref, src_ref, dst_ref, out_ref,
             acc_sh, deg_sh, g_hbm,
             src_v, dst_v, rowbuf, ones_v, zbuf, abuf, obuf,
             dinv_v, degbuf):
    c = lax.axis_index("c")
    s = lax.axis_index("s")

    for k in range(CH // 16):
        ones_v[pl.ds(k * 16, 16)] = jnp.full((16,), 1.0, jnp.float32)

    def _zbuf_fill(i, carry):
        for k in range(DH // 16):
            zbuf[i, pl.ds(k * 16, 16)] = jnp.zeros((16,), jnp.float32)
        return carry
    lax.fori_loop(0, ZR, _zbuf_fill, 0)

    def _degbuf_zero(j, carry):
        degbuf[pl.ds(j * 16, 16)] = jnp.zeros((16,), jnp.float32)
        return carry
    lax.fori_loop(0, NT // 16, _degbuf_zero, 0)

    pltpu.sync_copy(src_ref.at[s], src_v)
    pltpu.sync_copy(dst_ref.at[s], dst_v)

    pltpu.sync_copy(degbuf, deg_sh.at[pl.ds(s * NT, NT)])
    plsc.subcore_barrier()

    def _deg_body(j, carry):
        pltpu.sync_copy(ones_v, deg_sh.at[dst_v.at[j]], add=True)
        return carry
    lax.fori_loop(0, NCHUNK, _deg_body, 0)
    plsc.subcore_barrier()

    pltpu.sync_copy(deg_sh.at[pl.ds(s * NT, NT)], degbuf)

    def _dinv_body(j, carry):
        d = degbuf[pl.ds(j * 16, 16)]
        y = _rsqrt16(d)
        dinv_v[pl.ds(j * 16, 16)] = jnp.where(d > 0.5, y, 0.0)
        return carry
    lax.fori_loop(0, NT // 16, _dinv_body, 0)

    def _init_block(b, carry):
        base = s * NT + b * BR
        pltpu.sync_copy(x_ref.at[c, pl.ds(base, BR)], abuf)
        pltpu.sync_copy(abuf, out_ref.at[c, pl.ds(base, BR)])

        def _scale_init(g, carry2):
            dvec = dinv_v[pl.ds(b * BR + g * 16, 16)]
            for i in range(16):
                dv = dvec[i]
                r = g * 16 + i
                for k in range(DH // 16):
                    sl = pl.ds(k * 16, 16)
                    abuf[r, sl] = abuf[r, sl] * dv
            return carry2
        lax.fori_loop(0, BR // 16, _scale_init, 0)

        pltpu.sync_copy(abuf, g_hbm.at[c, pl.ds(base, BR)])
        pltpu.sync_copy(zbuf, acc_sh.at[pl.ds(base, ZR)])
        pltpu.sync_copy(zbuf, acc_sh.at[pl.ds(base + ZR, ZR)])
        return carry
    lax.fori_loop(0, NBLK, _init_block, 0)
    plsc.subcore_barrier()

    for ell in range(NLAYERS):
        last = ell == NLAYERS - 1

        def _edge_body(j, carry):
            pltpu.sync_copy(g_hbm.at[c].at[src_v.at[j]], rowbuf)
            pltpu.sync_copy(rowbuf, acc_sh.at[dst_v.at[j]], add=True)
            return carry
        lax.fori_loop(0, NCHUNK, _edge_body, 0)
        plsc.subcore_barrier()

        if not last:
            def _mid_block(b, carry):
                base = s * NT + b * BR
                pltpu.sync_copy(acc_sh.at[pl.ds(base, BR)], abuf)
                pltpu.sync_copy(out_ref.at[c, pl.ds(base, BR)], obuf)

                def _scale_mid(g, carry2):
                    dvec = dinv_v[pl.ds(b * BR + g * 16, 16)]
                    for i in range(16):
                        dv = dvec[i]
                        r = g * 16 + i
                        for k in range(DH // 16):
                            sl = pl.ds(k * 16, 16)
                            h = abuf[r, sl] * dv
                            abuf[r, sl] = h * dv
                            obuf[r, sl] = obuf[r, sl] + h
                    return carry2
                lax.fori_loop(0, BR // 16, _scale_mid, 0)

                pltpu.sync_copy(abuf, g_hbm.at[c, pl.ds(base, BR)])
                pltpu.sync_copy(obuf, out_ref.at[c, pl.ds(base, BR)])
                pltpu.sync_copy(zbuf, acc_sh.at[pl.ds(base, ZR)])
                pltpu.sync_copy(zbuf, acc_sh.at[pl.ds(base + ZR, ZR)])
                return carry
            lax.fori_loop(0, NBLK, _mid_block, 0)
            plsc.subcore_barrier()
        else:
            def _last_block(b, carry):
                base = s * NT + b * BR
                pltpu.sync_copy(acc_sh.at[pl.ds(base, BR)], abuf)
                pltpu.sync_copy(out_ref.at[c, pl.ds(base, BR)], obuf)

                def _scale_last(g, carry2):
                    dvec = dinv_v[pl.ds(b * BR + g * 16, 16)]
                    for i in range(16):
                        dv = dvec[i]
                        r = g * 16 + i
                        for k in range(DH // 16):
                            sl = pl.ds(k * 16, 16)
                            h = abuf[r, sl] * dv
                            obuf[r, sl] = (obuf[r, sl] + h) * ALPHA
                    return carry2
                lax.fori_loop(0, BR // 16, _scale_last, 0)

                pltpu.sync_copy(obuf, out_ref.at[c, pl.ds(base, BR)])
                return carry
            lax.fori_loop(0, NBLK, _last_block, 0)


_sc_kernel = functools.partial(
    pl.kernel,
    out_type=jax.ShapeDtypeStruct((2, NP, DH), jnp.float32),
    mesh=plsc.VectorSubcoreMesh(core_axis_name="c", subcore_axis_name="s"),
    compiler_params=pltpu.CompilerParams(use_tc_tiling_on_sc=False),
    scratch_types=[
        pltpu.VMEM_SHARED((NP, DH), jnp.float32),
        pltpu.VMEM_SHARED((NP,), jnp.float32),
        pltpu.HBM((2, NP, DH), jnp.float32),
        pltpu.VMEM((NCHUNK, CH), jnp.int32),
        pltpu.VMEM((NCHUNK, CH), jnp.int32),
        pltpu.VMEM((CH, DH), jnp.float32),
        pltpu.VMEM((CH,), jnp.float32),
        pltpu.VMEM((ZR, DH), jnp.float32),
        pltpu.VMEM((BR, DH), jnp.float32),
        pltpu.VMEM((BR, DH), jnp.float32),
        pltpu.VMEM((NT,), jnp.float32),
        pltpu.VMEM((NT,), jnp.float32),
    ],
)(_sc_body)


@jax.jit
def kernel(x, edge_index):
    xp = jnp.zeros((NP, D), jnp.float32).at[:N].set(x)
    x_cm = xp.reshape(NP, 2, DH).transpose(1, 0, 2)
    src_r = edge_index[0].reshape(NSUB, NCHUNK, CH)
    dst_r = edge_index[1].reshape(NSUB, NCHUNK, CH)
    out = _sc_kernel(x_cm, src_r, dst_r)
    return out.transpose(1, 0, 2).reshape(NP, D)[:N]


# speedup vs baseline: 15.4483x; 1.6493x over previous
import functools

import jax
import jax.numpy as jnp
from jax import lax
from jax.experimental import pallas as pl
from jax.experimental.pallas import tpu as pltpu
from jax.experimental.pallas import tpu_sc as plsc

N = 10000
NP = 10240
E = 320000
D = 128
DH = 64
NSUB = 16
NT = NP // NSUB
ET = E // NSUB
CH = 80
NCHUNK = ET // CH
BR = 80
NBLK = NT // BR
ZR = 40
NLAYERS = 3
ALPHA = 1.0 / (NLAYERS + 1)


def _rsqrt16(d):
    y = jnp.full((16,), 1.0, jnp.float32)
    for k in range(1, 20):
        y = jnp.where(d >= float(2 ** k), float(2.0 ** (-k / 2.0)), y)
    for _ in range(4):
        y = y * (1.5 - (0.5 * d) * y * y)
    return y


def _sc_body(x_ref, src_ref, dst_ref, out_ref,
             acc_sh, deg_sh, g_hbm,
             src_v, dst_v, rowbuf, rowbuf2, ones_v, zbuf, abuf, obuf,
             dinv_v, degbuf, gsem0, gsem1):
    c = lax.axis_index("c")
    s = lax.axis_index("s")

    for k in range(CH // 16):
        ones_v[pl.ds(k * 16, 16)] = jnp.full((16,), 1.0, jnp.float32)

    def _zbuf_fill(i, carry):
        for k in range(DH // 16):
            zbuf[i, pl.ds(k * 16, 16)] = jnp.zeros((16,), jnp.float32)
        return carry
    lax.fori_loop(0, ZR, _zbuf_fill, 0)

    def _degbuf_zero(j, carry):
        degbuf[pl.ds(j * 16, 16)] = jnp.zeros((16,), jnp.float32)
        return carry
    lax.fori_loop(0, NT // 16, _degbuf_zero, 0)

    pltpu.sync_copy(src_ref.at[s], src_v)
    pltpu.sync_copy(dst_ref.at[s], dst_v)

    pltpu.sync_copy(degbuf, deg_sh.at[pl.ds(s * NT, NT)])
    plsc.subcore_barrier()

    def _deg_body(j, carry):
        pltpu.sync_copy(ones_v, deg_sh.at[dst_v.at[j]], add=True)
        return carry
    lax.fori_loop(0, NCHUNK, _deg_body, 0)
    plsc.subcore_barrier()

    pltpu.sync_copy(deg_sh.at[pl.ds(s * NT, NT)], degbuf)

    def _dinv_body(j, carry):
        d = degbuf[pl.ds(j * 16, 16)]
        y = _rsqrt16(d)
        dinv_v[pl.ds(j * 16, 16)] = jnp.where(d > 0.5, y, 0.0)
        return carry
    lax.fori_loop(0, NT // 16, _dinv_body, 0)

    def _init_block(b, carry):
        base = s * NT + b * BR
        pltpu.sync_copy(x_ref.at[c, pl.ds(base, BR)], abuf)
        pltpu.sync_copy(abuf, out_ref.at[c, pl.ds(base, BR)])

        def _scale_init(g, carry2):
            dvec = dinv_v[pl.ds(b * BR + g * 16, 16)]
            for i in range(16):
                dv = dvec[i]
                r = g * 16 + i
                for k in range(DH // 16):
                    sl = pl.ds(k * 16, 16)
                    abuf[r, sl] = abuf[r, sl] * dv
            return carry2
        lax.fori_loop(0, BR // 16, _scale_init, 0)

        pltpu.sync_copy(abuf, g_hbm.at[c, pl.ds(base, BR)])
        pltpu.sync_copy(zbuf, acc_sh.at[pl.ds(base, ZR)])
        pltpu.sync_copy(zbuf, acc_sh.at[pl.ds(base + ZR, ZR)])
        return carry
    lax.fori_loop(0, NBLK, _init_block, 0)
    plsc.subcore_barrier()

    for ell in range(NLAYERS):
        last = ell == NLAYERS - 1

        pltpu.async_copy(g_hbm.at[c].at[src_v.at[0]], rowbuf, gsem0)

        def _pair_body(jj, carry):
            j0 = jj * 2
            j1 = j0 + 1
            j2 = lax.rem(j0 + 2, NCHUNK)
            pltpu.async_copy(g_hbm.at[c].at[src_v.at[j1]], rowbuf2, gsem1)
            pltpu.make_async_copy(
                g_hbm.at[c].at[src_v.at[j0]], rowbuf, gsem0).wait()
            pltpu.sync_copy(rowbuf, acc_sh.at[dst_v.at[j0]], add=True)
            pltpu.async_copy(g_hbm.at[c].at[src_v.at[j2]], rowbuf, gsem0)
            pltpu.make_async_copy(
                g_hbm.at[c].at[src_v.at[j1]], rowbuf2, gsem1).wait()
            pltpu.sync_copy(rowbuf2, acc_sh.at[dst_v.at[j1]], add=True)
            return carry
        lax.fori_loop(0, NCHUNK // 2, _pair_body, 0)
        pltpu.make_async_copy(
            g_hbm.at[c].at[src_v.at[0]], rowbuf, gsem0).wait()
        plsc.subcore_barrier()

        if not last:
            def _mid_block(b, carry):
                base = s * NT + b * BR
                pltpu.sync_copy(acc_sh.at[pl.ds(base, BR)], abuf)
                pltpu.sync_copy(out_ref.at[c, pl.ds(base, BR)], obuf)

                def _scale_mid(g, carry2):
                    dvec = dinv_v[pl.ds(b * BR + g * 16, 16)]
                    for i in range(16):
                        dv = dvec[i]
                        r = g * 16 + i
                        for k in range(DH // 16):
                            sl = pl.ds(k * 16, 16)
                            h = abuf[r, sl] * dv
                            abuf[r, sl] = h * dv
                            obuf[r, sl] = obuf[r, sl] + h
                    return carry2
                lax.fori_loop(0, BR // 16, _scale_mid, 0)

                pltpu.sync_copy(abuf, g_hbm.at[c, pl.ds(base, BR)])
                pltpu.sync_copy(obuf, out_ref.at[c, pl.ds(base, BR)])
                pltpu.sync_copy(zbuf, acc_sh.at[pl.ds(base, ZR)])
                pltpu.sync_copy(zbuf, acc_sh.at[pl.ds(base + ZR, ZR)])
                return carry
            lax.fori_loop(0, NBLK, _mid_block, 0)
            plsc.subcore_barrier()
        else:
            def _last_block(b, carry):
                base = s * NT + b * BR
                pltpu.sync_copy(acc_sh.at[pl.ds(base, BR)], abuf)
                pltpu.sync_copy(out_ref.at[c, pl.ds(base, BR)], obuf)

                def _scale_last(g, carry2):
                    dvec = dinv_v[pl.ds(b * BR + g * 16, 16)]
                    for i in range(16):
                        dv = dvec[i]
                        r = g * 16 + i
                        for k in range(DH // 16):
                            sl = pl.ds(k * 16, 16)
                            h = abuf[r, sl] * dv
                            obuf[r, sl] = (obuf[r, sl] + h) * ALPHA
                    return carry2
                lax.fori_loop(0, BR // 16, _scale_last, 0)

                pltpu.sync_copy(obuf, out_ref.at[c, pl.ds(base, BR)])
                return carry
            lax.fori_loop(0, NBLK, _last_block, 0)


_sc_kernel = functools.partial(
    pl.kernel,
    out_type=jax.ShapeDtypeStruct((2, NP, DH), jnp.float32),
    mesh=plsc.VectorSubcoreMesh(core_axis_name="c", subcore_axis_name="s"),
    compiler_params=pltpu.CompilerParams(use_tc_tiling_on_sc=False),
    scratch_types=[
        pltpu.VMEM_SHARED((NP, DH), jnp.float32),
        pltpu.VMEM_SHARED((NP,), jnp.float32),
        pltpu.HBM((2, NP, DH), jnp.float32),
        pltpu.VMEM((NCHUNK, CH), jnp.int32),
        pltpu.VMEM((NCHUNK, CH), jnp.int32),
        pltpu.VMEM((CH, DH), jnp.float32),
        pltpu.VMEM((CH, DH), jnp.float32),
        pltpu.VMEM((CH,), jnp.float32),
        pltpu.VMEM((ZR, DH), jnp.float32),
        pltpu.VMEM((BR, DH), jnp.float32),
        pltpu.VMEM((BR, DH), jnp.float32),
        pltpu.VMEM((NT,), jnp.float32),
        pltpu.VMEM((NT,), jnp.float32),
        pltpu.SemaphoreType.DMA,
        pltpu.SemaphoreType.DMA,
    ],
)(_sc_body)


@jax.jit
def kernel(x, edge_index):
    xp = jnp.zeros((NP, D), jnp.float32).at[:N].set(x)
    x_cm = xp.reshape(NP, 2, DH).transpose(1, 0, 2)
    src_r = edge_index[0].reshape(NSUB, NCHUNK, CH)
    dst_r = edge_index[1].reshape(NSUB, NCHUNK, CH)
    out = _sc_kernel(x_cm, src_r, dst_r)
    return out.transpose(1, 0, 2).reshape(NP, D)[:N]
